# baseline (speedup 1.0000x reference)
import functools

import jax
import jax.numpy as jnp
from jax import lax
from jax.experimental import pallas as pl
from jax.experimental.pallas import tpu as pltpu
from jax.experimental.pallas import tpu_sc as plsc

_NC = 2
_NS = 16
_NW = _NC * _NS

_B = 4096
_S = 200
_N = _B * _S
_WD = 50
_PD = 5
_OD = _WD + 2 * _PD
_WDP = 56
_PDP = 8

_C = _S
_SPLITS = ((0, 128), (128, 72))
_NPW = _N // _NW
_NCHUNK = _NPW // _C
_SUP = 8
_NSUP = _NCHUNK // _SUP
_U = 4


def _ivec(vals):
    lanes = lax.broadcasted_iota(jnp.int32, (16,), 0)
    out = lanes * 0
    for i, x in enumerate(vals):
        out = jnp.where(lanes == i, jnp.int32(x), out)
    return out


def _make_body():
    def body(word_hbm, pos1_hbm, pos2_hbm, wtab_hbm, p1tab_hbm, p2tab_hbm,
             out_hbm,
             sidxw, sidxp1, sidxp2,
             wrow0, pbuf0, outv0, wrow1, pbuf1, outv1,
             gsem0, gsem1, osem0, osem1):
        wid = lax.axis_index("s") * _NC + lax.axis_index("c")
        row0 = wid * _NPW
        lanes = lax.broadcasted_iota(jnp.int32, (16,), 0)
        w_col = _ivec([44, 45, 46, 47, 48, 49] + [0] * 10)
        pos_rowoff = _ivec([0] * 11 + [_C] * 5)
        pos_col = _ivec([0] * 6 + [0, 1, 2, 3, 4, 0, 1, 2, 3, 4])
        m_w = lanes < 6
        zeros = lanes * 0

        bufs = ((wrow0, pbuf0, outv0, gsem0, osem0),
                (wrow1, pbuf1, outv1, gsem1, osem1))

        def load_idx(si):
            base = row0 + si * _SUP * _C
            pltpu.sync_copy(word_hbm.at[pl.ds(base, _SUP * _C)], sidxw)
            pltpu.sync_copy(pos1_hbm.at[pl.ds(base, _SUP * _C)], sidxp1)
            pltpu.sync_copy(pos2_hbm.at[pl.ds(base, _SUP * _C)], sidxp2)

        def fire(cis, b):
            wrow, pbuf, _, gsem, _ = bufs[b]
            for so, sn in _SPLITS:
                off = cis * _C + so
                rows = pl.ds(so, sn)
                pltpu.async_copy(
                    wtab_hbm.at[sidxw.at[pl.ds(off, sn)]],
                    wrow.at[rows], gsem)
                pltpu.async_copy(
                    p1tab_hbm.at[sidxp1.at[pl.ds(off, sn)]],
                    pbuf.at[rows], gsem)
                pltpu.async_copy(
                    p2tab_hbm.at[sidxp2.at[pl.ds(off, sn)]],
                    pbuf.at[pl.ds(_C + so, sn)], gsem)

        def wait_gathers(cis, b):
            wrow, pbuf, _, gsem, _ = bufs[b]
            for so, sn in _SPLITS:
                off = cis * _C + so
                rows = pl.ds(so, sn)
                pltpu.make_async_copy(
                    wtab_hbm.at[sidxw.at[pl.ds(off, sn)]],
                    wrow.at[rows], gsem).wait()
                pltpu.make_async_copy(
                    p1tab_hbm.at[sidxp1.at[pl.ds(off, sn)]],
                    pbuf.at[rows], gsem).wait()
                pltpu.make_async_copy(
                    p2tab_hbm.at[sidxp2.at[pl.ds(off, sn)]],
                    pbuf.at[pl.ds(_C + so, sn)], gsem).wait()

        def assemble(b):
            wrow, pbuf, outv, _, _ = bufs[b]

            def rows_body(g, _):
                r_base = g * _U
                for u in range(_U):
                    r = r_base + u
                    outv[r, pl.ds(0, 16)] = wrow[r, pl.ds(0, 16)]
                    outv[r, pl.ds(16, 16)] = wrow[r, pl.ds(16, 16)]
                    outv[r, pl.ds(28, 16)] = wrow[r, pl.ds(28, 16)]
                    rv = zeros + r
                    gw = plsc.load_gather(wrow, [rv, w_col])
                    gp = plsc.load_gather(pbuf, [rv + pos_rowoff, pos_col])
                    outv[r, pl.ds(44, 16)] = jnp.where(m_w, gw, gp)
                return ()

            lax.fori_loop(0, _C // _U, rows_body, ())

        batch0 = wid * (_NPW // _S)

        def write_out(cg, b):
            _, _, outv, _, osem = bufs[b]
            pltpu.async_copy(outv, out_hbm.at[batch0 + cg], osem)

        def wait_out(cg, b):
            _, _, outv, _, osem = bufs[b]
            pltpu.make_async_copy(
                outv, out_hbm.at[batch0 + cg], osem).wait()

        def sup_body(si, _):
            load_idx(si)
            fire(0, 0)
            for pj in range(_SUP // 2):
                cg0 = si * _SUP + pj * 2
                wait_gathers(pj * 2, 0)
                fire(pj * 2 + 1, 1)
                lax.cond(cg0 >= 2, lambda: wait_out(cg0 - 2, 0),
                         lambda: None)
                assemble(0)
                write_out(cg0, 0)
                wait_gathers(pj * 2 + 1, 1)
                if pj < _SUP // 2 - 1:
                    fire(pj * 2 + 2, 0)
                lax.cond(cg0 >= 1, lambda: wait_out(cg0 - 1, 1),
                         lambda: None)
                assemble(1)
                write_out(cg0 + 1, 1)
            return ()

        lax.fori_loop(0, _NSUP, sup_body, ())
        wait_out(_NCHUNK - 2, 0)
        wait_out(_NCHUNK - 1, 1)

    return body


@jax.jit
def _embed(word, pos1, pos2, word_table, pos1_table, pos2_table):
    mesh = plsc.VectorSubcoreMesh(
        core_axis_name="c", subcore_axis_name="s",
        num_cores=_NC, num_subcores=_NS)
    run = pl.kernel(
        _make_body(),
        out_type=jax.ShapeDtypeStruct((_B, _S, _OD), jnp.float32),
        mesh=mesh,
        scratch_types=[
            pltpu.VMEM((_SUP * _C,), jnp.int32),
            pltpu.VMEM((_SUP * _C,), jnp.int32),
            pltpu.VMEM((_SUP * _C,), jnp.int32),
            pltpu.VMEM((_C, _WDP), jnp.float32),
            pltpu.VMEM((2 * _C, _PDP), jnp.float32),
            pltpu.VMEM((_C, _OD), jnp.float32),
            pltpu.VMEM((_C, _WDP), jnp.float32),
            pltpu.VMEM((2 * _C, _PDP), jnp.float32),
            pltpu.VMEM((_C, _OD), jnp.float32),
            pltpu.SemaphoreType.DMA,
            pltpu.SemaphoreType.DMA,
            pltpu.SemaphoreType.DMA,
            pltpu.SemaphoreType.DMA,
        ],
        compiler_params=pltpu.CompilerParams(
            use_tc_tiling_on_sc=False, needs_layout_passes=False),
    )
    wtab = jnp.pad(word_table, ((0, 0), (0, _WDP - _WD)))
    p1tab = jnp.pad(pos1_table, ((0, 0), (0, _PDP - _PD)))
    p2tab = jnp.pad(pos2_table, ((0, 0), (0, _PDP - _PD)))
    return run(word.reshape(_N), pos1.reshape(_N), pos2.reshape(_N),
               wtab, p1tab, p2tab)


def kernel(word, pos1, pos2, word_table, pos1_table, pos2_table):
    return _embed(word, pos1, pos2, word_table, pos1_table, pos2_table)
